# Initial kernel scaffold; baseline (speedup 1.0000x reference)
import jax
import jax.numpy as jnp
from jax.experimental import pallas as pl


def kernel(x, edge_index, W, bias, gamma, beta):
    raise NotImplementedError("write your pallas kernel here")


# speedup vs baseline: 6.1847x; 6.1847x over previous
import functools

import jax
import jax.numpy as jnp
from jax import lax
from jax.experimental import pallas as pl
from jax.experimental.pallas import tpu as pltpu
from jax.experimental.pallas import tpu_sc as plsc

NC = 2
NS = 16
N = 10000
E = 160000
C = 256
HALF = C // 2
NPAD = 10240
ZPT = NPAD // NS
CH = 128
EPS = 1e-5

_MESH = plsc.VectorSubcoreMesh(core_axis_name="c", subcore_axis_name="s",
                               num_cores=NC, num_subcores=NS)


DW = 128


def _deg_body(rowi, ones_h, zrows, out, acc, ones_v, ones_t, ridx_v, ridx_t):
    c = lax.axis_index("c")
    s = lax.axis_index("s")
    pltpu.sync_copy(zrows, acc.at[pl.ds(s * ZPT, ZPT)])
    pltpu.sync_copy(ones_h, ones_v)
    pltpu.sync_copy(ones_h.at[pl.ds(0, 8)], ones_t)
    plsc.subcore_barrier()

    ept = E // (NC * NS)
    full = ept // CH
    tail = ept - full * CH
    ebase = (c * NS + s) * ept

    def do(off, ridx, ones, sz):
        pltpu.sync_copy(rowi.at[pl.ds(off, sz)], ridx)
        pltpu.sync_copy(ones, acc.at[ridx], add=True)

    def body(i, carry):
        do(ebase + i * CH, ridx_v, ones_v, CH)
        return carry

    lax.fori_loop(0, full, body, 0)
    do(ebase + full * CH, ridx_t, ones_t, tail)
    plsc.subcore_barrier()
    pltpu.sync_copy(acc.at[pl.ds(s * ZPT, ZPT)], out.at[c, pl.ds(s * ZPT, ZPT)])


def _deg_call(row):
    ones_h = jnp.ones((CH, DW), jnp.float32)
    zrows = jnp.zeros((ZPT, DW), jnp.float32)
    fn = pl.kernel(
        _deg_body,
        out_type=jax.ShapeDtypeStruct((NC, NPAD, DW), jnp.float32),
        mesh=_MESH,
        scratch_types=[
            pltpu.VMEM_SHARED((NPAD, DW), jnp.float32),
            pltpu.VMEM((CH, DW), jnp.float32),
            pltpu.VMEM((8, DW), jnp.float32),
            pltpu.VMEM((CH,), jnp.int32),
            pltpu.VMEM((8,), jnp.int32),
        ],
    )
    return fn(row, ones_h, zrows)


def _prop_body(hp, rowi, coli, zrows, out, acc,
               rows_v, rows_t, ridx_v, ridx_t, cidx_v, cidx_t, sem):
    c = lax.axis_index("c")
    s = lax.axis_index("s")
    coff = c * NPAD
    pltpu.sync_copy(zrows, acc.at[pl.ds(s * ZPT, ZPT)])
    plsc.subcore_barrier()

    ept = E // NS
    full = ept // CH
    tail = ept - full * CH
    ebase = s * ept

    def do(off, ridx, cidx, rows, sz):
        pltpu.sync_copy(rowi.at[pl.ds(off, sz)], ridx)
        pltpu.sync_copy(coli.at[pl.ds(off, sz)], cidx)
        for k in range(sz // 16):
            sl = pl.ds(k * 16, 16)
            ridx[sl] = ridx[sl] + coff
        pltpu.async_copy(hp.at[ridx], rows, sem).wait()
        pltpu.sync_copy(rows, acc.at[cidx], add=True)

    def body(i, carry):
        do(ebase + i * CH, ridx_v, cidx_v, rows_v, CH)
        return carry

    lax.fori_loop(0, full, body, 0)
    do(ebase + full * CH, ridx_t, cidx_t, rows_t, tail)
    plsc.subcore_barrier()
    pltpu.sync_copy(acc.at[pl.ds(s * ZPT, ZPT)], out.at[c, pl.ds(s * ZPT, ZPT)])


def _prop_call(hp_flat, row, col):
    zrows = jnp.zeros((ZPT, HALF), jnp.float32)
    fn = pl.kernel(
        _prop_body,
        out_type=jax.ShapeDtypeStruct((NC, NPAD, HALF), jnp.float32),
        mesh=_MESH,
        scratch_types=[
            pltpu.VMEM_SHARED((NPAD, HALF), jnp.float32),
            pltpu.VMEM((CH, HALF), jnp.float32),
            pltpu.VMEM((16, HALF), jnp.float32),
            pltpu.VMEM((CH,), jnp.int32),
            pltpu.VMEM((16,), jnp.int32),
            pltpu.VMEM((CH,), jnp.int32),
            pltpu.VMEM((16,), jnp.int32),
            pltpu.SemaphoreType.DMA,
        ],
    )
    return fn(hp_flat, row, col, zrows)


_BN = 400
_NB = N // _BN


def _diw_block(deg_ref):
    deg = jnp.sum(deg_ref[...], axis=1, keepdims=True)
    return jnp.where(deg > 0, lax.rsqrt(jnp.maximum(deg, 1.0)), 0.0)


def _e1_body(deg_ref, x_ref, out_ref):
    d2 = _diw_block(deg_ref)
    out_ref[0] = d2 * x_ref[:, :HALF]
    out_ref[1] = d2 * x_ref[:, HALF:]


def _e1_call(deg2, x):
    return pl.pallas_call(
        _e1_body,
        grid=(_NB,),
        in_specs=[pl.BlockSpec((_BN, 2), lambda i: (i, 0)),
                  pl.BlockSpec((_BN, C), lambda i: (i, 0))],
        out_specs=pl.BlockSpec((NC, _BN, HALF), lambda i: (0, i, 0)),
        out_shape=jax.ShapeDtypeStruct((NC, NPAD, HALF), jnp.float32),
    )(deg2, x)


def _e2_body(deg_ref, a_ref, out_ref):
    deg = jnp.sum(deg_ref[...], axis=1, keepdims=True)
    w = jnp.where(deg > 0, -1.0 / jnp.maximum(deg, 1.0), 0.0)
    out_ref[0] = w * a_ref[0]
    out_ref[1] = w * a_ref[1]


def _e2_call(deg2, acc1):
    return pl.pallas_call(
        _e2_body,
        grid=(_NB,),
        in_specs=[pl.BlockSpec((_BN, 2), lambda i: (i, 0)),
                  pl.BlockSpec((NC, _BN, HALF), lambda i: (0, i, 0))],
        out_specs=pl.BlockSpec((NC, _BN, HALF), lambda i: (0, i, 0)),
        out_shape=jax.ShapeDtypeStruct((NC, NPAD, HALF), jnp.float32),
    )(deg2, acc1)


def _d1_body(deg_ref, x_ref, a1_ref, a2_ref, w_ref, b_ref,
             y_ref, st_ref, s_acc, q_acc):
    i = pl.program_id(0)
    d2 = _diw_block(deg_ref)
    x = x_ref[...]
    t1 = -d2 * jnp.concatenate([a1_ref[0], a1_ref[1]], axis=1)
    t2 = -2.0 * d2 * jnp.concatenate([a2_ref[0], a2_ref[1]], axis=1) - x
    y = jnp.dot(x, w_ref[0], preferred_element_type=jnp.float32)
    y = y + jnp.dot(t1, w_ref[1], preferred_element_type=jnp.float32)
    y = y + jnp.dot(t2, w_ref[2], preferred_element_type=jnp.float32)
    y = y + b_ref[...][None, :]
    y_ref[...] = y

    @pl.when(i == 0)
    def _():
        s_acc[...] = jnp.zeros_like(s_acc)
        q_acc[...] = jnp.zeros_like(q_acc)

    s_acc[...] += jnp.sum(y, axis=0, keepdims=True)
    q_acc[...] += jnp.sum(y * y, axis=0, keepdims=True)

    @pl.when(i == _NB - 1)
    def _():
        st_ref[0] = s_acc[0]
        st_ref[1] = q_acc[0]


def _d1_call(deg2, x, acc1, acc2, W, bias):
    return pl.pallas_call(
        _d1_body,
        grid=(_NB,),
        in_specs=[pl.BlockSpec((_BN, 2), lambda i: (i, 0)),
                  pl.BlockSpec((_BN, C), lambda i: (i, 0)),
                  pl.BlockSpec((NC, _BN, HALF), lambda i: (0, i, 0)),
                  pl.BlockSpec((NC, _BN, HALF), lambda i: (0, i, 0)),
                  pl.BlockSpec((3, C, C), lambda i: (0, 0, 0)),
                  pl.BlockSpec((C,), lambda i: (0,))],
        out_specs=[pl.BlockSpec((_BN, C), lambda i: (i, 0)),
                   pl.BlockSpec((2, C), lambda i: (0, 0))],
        out_shape=[jax.ShapeDtypeStruct((N, C), jnp.float32),
                   jax.ShapeDtypeStruct((2, C), jnp.float32)],
        scratch_shapes=[pltpu.VMEM((1, C), jnp.float32),
                        pltpu.VMEM((1, C), jnp.float32)],
    )(deg2, x, acc1, acc2, W, bias)


def _d2_body(st_ref, x_ref, y_ref, g_ref, bt_ref, o_ref):
    mean = st_ref[0] / N
    var = st_ref[1] / N - mean * mean
    scale = (lax.rsqrt(var + EPS) * g_ref[...])[None, :]
    o_ref[...] = jnp.maximum(
        (y_ref[...] - mean[None, :]) * scale + bt_ref[...][None, :] + x_ref[...],
        0.0)


def _d2_call(st, x, y, gamma, beta):
    return pl.pallas_call(
        _d2_body,
        grid=(_NB,),
        in_specs=[pl.BlockSpec((2, C), lambda i: (0, 0)),
                  pl.BlockSpec((_BN, C), lambda i: (i, 0)),
                  pl.BlockSpec((_BN, C), lambda i: (i, 0)),
                  pl.BlockSpec((C,), lambda i: (0,)),
                  pl.BlockSpec((C,), lambda i: (0,))],
        out_specs=pl.BlockSpec((_BN, C), lambda i: (i, 0)),
        out_shape=jax.ShapeDtypeStruct((N, C), jnp.float32),
    )(st, x, y, gamma, beta)


def kernel(x, edge_index, W, bias, gamma, beta):
    row = edge_index[0]
    col = edge_index[1]
    degs = _deg_call(row)
    deg2 = degs[:, :N, 0].T
    hp0 = _e1_call(deg2, x).reshape(NC * NPAD, HALF)
    acc1 = _prop_call(hp0, row, col)
    hp1 = _e2_call(deg2, acc1).reshape(NC * NPAD, HALF)
    acc2 = _prop_call(hp1, row, col)
    y, st = _d1_call(deg2, x, acc1, acc2, W, bias)
    return _d2_call(st, x, y, gamma, beta)


# speedup vs baseline: 8.3058x; 1.3430x over previous
import functools

import jax
import jax.numpy as jnp
from jax import lax
from jax.experimental import pallas as pl
from jax.experimental.pallas import tpu as pltpu
from jax.experimental.pallas import tpu_sc as plsc

NC = 2
NS = 16
N = 10000
E = 160000
C = 256
HALF = C // 2
NPAD = 10240
ZPT = NPAD // NS
CH = 128
EPS = 1e-5

_MESH = plsc.VectorSubcoreMesh(core_axis_name="c", subcore_axis_name="s",
                               num_cores=NC, num_subcores=NS)


DW = 128


def _deg_body(rowi, ones_h, zrows, out, acc, ones_v, ones_t, ridx_v, ridx_t):
    c = lax.axis_index("c")
    s = lax.axis_index("s")
    pltpu.sync_copy(zrows, acc.at[pl.ds(s * ZPT, ZPT)])
    pltpu.sync_copy(ones_h, ones_v)
    pltpu.sync_copy(ones_h.at[pl.ds(0, 8)], ones_t)
    plsc.subcore_barrier()

    ept = E // (NC * NS)
    full = ept // CH
    tail = ept - full * CH
    ebase = (c * NS + s) * ept

    def do(off, ridx, ones, sz):
        pltpu.sync_copy(rowi.at[pl.ds(off, sz)], ridx)
        pltpu.sync_copy(ones, acc.at[ridx], add=True)

    def body(i, carry):
        do(ebase + i * CH, ridx_v, ones_v, CH)
        return carry

    lax.fori_loop(0, full, body, 0)
    do(ebase + full * CH, ridx_t, ones_t, tail)
    plsc.subcore_barrier()
    pltpu.sync_copy(acc.at[pl.ds(s * ZPT, ZPT)], out.at[c, pl.ds(s * ZPT, ZPT)])


def _deg_call(row):
    ones_h = jnp.ones((CH, DW), jnp.float32)
    zrows = jnp.zeros((ZPT, DW), jnp.float32)
    fn = pl.kernel(
        _deg_body,
        out_type=jax.ShapeDtypeStruct((NC, NPAD, DW), jnp.float32),
        mesh=_MESH,
        scratch_types=[
            pltpu.VMEM_SHARED((NPAD, DW), jnp.float32),
            pltpu.VMEM((CH, DW), jnp.float32),
            pltpu.VMEM((8, DW), jnp.float32),
            pltpu.VMEM((CH,), jnp.int32),
            pltpu.VMEM((8,), jnp.int32),
        ],
    )
    return fn(row, ones_h, zrows)


_DEPTH = 2


def _prop_body(hp, rowi, coli, zrows, out, acc,
               rows3, ridx3, cidx3, rows_t, ridx_t, cidx_t,
               gsem3, ssem3, gsem_t):
    c = lax.axis_index("c")
    s = lax.axis_index("s")
    coff = c * NPAD
    pltpu.sync_copy(zrows, acc.at[pl.ds(s * ZPT, ZPT)])
    plsc.subcore_barrier()

    ept = E // NS
    full = ept // CH
    tail = ept - full * CH
    npair = full // _DEPTH
    ebase = s * ept

    def loadidx(off, ridx, cidx, sz):
        pltpu.sync_copy(rowi.at[pl.ds(off, sz)], ridx)
        pltpu.sync_copy(coli.at[pl.ds(off, sz)], cidx)
        for k in range(sz // 16):
            sl = pl.ds(k * 16, 16)
            ridx[sl] = ridx[sl] + coff

    def stage(d):
        return rows3[d], ridx3[d], cidx3[d], gsem3[d], ssem3[d]

    for d in range(_DEPTH):
        rows, ridx, cidx, gsem, _ = stage(d)
        loadidx(ebase + d * CH, ridx, cidx, CH)
        pltpu.async_copy(hp.at[ridx], rows, gsem)

    def body(j, carry):
        base_next = ebase + (j + 1) * _DEPTH * CH
        for d in range(_DEPTH):
            rows, ridx, cidx, gsem, ssem = stage(d)
            pltpu.make_async_copy(hp.at[ridx], rows, gsem).wait()
            pltpu.async_copy(rows, acc.at[cidx], ssem, add=True)

        @pl.when(j < npair - 1)
        def _():
            for d in range(_DEPTH):
                rows, ridx, cidx, gsem, ssem = stage(d)
                pltpu.make_async_copy(rows, acc.at[cidx], ssem).wait()
                loadidx(base_next + d * CH, ridx, cidx, CH)
                pltpu.async_copy(hp.at[ridx], rows, gsem)

        return carry

    lax.fori_loop(0, npair, body, 0)
    for d in range(_DEPTH):
        rows, ridx, cidx, _, ssem = stage(d)
        pltpu.make_async_copy(rows, acc.at[cidx], ssem).wait()

    loadidx(ebase + full * CH, ridx_t, cidx_t, tail)
    pltpu.async_copy(hp.at[ridx_t], rows_t, gsem_t).wait()
    pltpu.sync_copy(rows_t, acc.at[cidx_t], add=True)

    plsc.subcore_barrier()
    pltpu.sync_copy(acc.at[pl.ds(s * ZPT, ZPT)], out.at[c, pl.ds(s * ZPT, ZPT)])


def _prop_call(hp_flat, row, col):
    zrows = jnp.zeros((ZPT, HALF), jnp.float32)
    fn = pl.kernel(
        _prop_body,
        out_type=jax.ShapeDtypeStruct((NC, NPAD, HALF), jnp.float32),
        mesh=_MESH,
        scratch_types=[
            pltpu.VMEM_SHARED((NPAD, HALF), jnp.float32),
            [pltpu.VMEM((CH, HALF), jnp.float32)] * _DEPTH,
            [pltpu.VMEM((CH,), jnp.int32)] * _DEPTH,
            [pltpu.VMEM((CH,), jnp.int32)] * _DEPTH,
            pltpu.VMEM((16, HALF), jnp.float32),
            pltpu.VMEM((16,), jnp.int32),
            pltpu.VMEM((16,), jnp.int32),
            [pltpu.SemaphoreType.DMA] * _DEPTH,
            [pltpu.SemaphoreType.DMA] * _DEPTH,
            pltpu.SemaphoreType.DMA,
        ],
    )
    return fn(hp_flat, row, col, zrows)


_BN = 400
_NB = N // _BN


def _diw_block(deg_ref):
    deg = jnp.sum(deg_ref[...], axis=1, keepdims=True)
    return jnp.where(deg > 0, lax.rsqrt(jnp.maximum(deg, 1.0)), 0.0)


def _e1_body(deg_ref, x_ref, out_ref):
    d2 = _diw_block(deg_ref)
    out_ref[0] = d2 * x_ref[:, :HALF]
    out_ref[1] = d2 * x_ref[:, HALF:]


def _e1_call(deg2, x):
    return pl.pallas_call(
        _e1_body,
        grid=(_NB,),
        in_specs=[pl.BlockSpec((_BN, 2), lambda i: (i, 0)),
                  pl.BlockSpec((_BN, C), lambda i: (i, 0))],
        out_specs=pl.BlockSpec((NC, _BN, HALF), lambda i: (0, i, 0)),
        out_shape=jax.ShapeDtypeStruct((NC, NPAD, HALF), jnp.float32),
    )(deg2, x)


def _e2_body(deg_ref, a_ref, out_ref):
    deg = jnp.sum(deg_ref[...], axis=1, keepdims=True)
    w = jnp.where(deg > 0, -1.0 / jnp.maximum(deg, 1.0), 0.0)
    out_ref[0] = w * a_ref[0]
    out_ref[1] = w * a_ref[1]


def _e2_call(deg2, acc1):
    return pl.pallas_call(
        _e2_body,
        grid=(_NB,),
        in_specs=[pl.BlockSpec((_BN, 2), lambda i: (i, 0)),
                  pl.BlockSpec((NC, _BN, HALF), lambda i: (0, i, 0))],
        out_specs=pl.BlockSpec((NC, _BN, HALF), lambda i: (0, i, 0)),
        out_shape=jax.ShapeDtypeStruct((NC, NPAD, HALF), jnp.float32),
    )(deg2, acc1)


def _d1_body(deg_ref, x_ref, a1_ref, a2_ref, w_ref, b_ref,
             y_ref, st_ref, s_acc, q_acc):
    i = pl.program_id(0)
    d2 = _diw_block(deg_ref)
    x = x_ref[...]
    t1 = -d2 * jnp.concatenate([a1_ref[0], a1_ref[1]], axis=1)
    t2 = -2.0 * d2 * jnp.concatenate([a2_ref[0], a2_ref[1]], axis=1) - x
    y = jnp.dot(x, w_ref[0], preferred_element_type=jnp.float32)
    y = y + jnp.dot(t1, w_ref[1], preferred_element_type=jnp.float32)
    y = y + jnp.dot(t2, w_ref[2], preferred_element_type=jnp.float32)
    y = y + b_ref[...][None, :]
    y_ref[...] = y

    @pl.when(i == 0)
    def _():
        s_acc[...] = jnp.zeros_like(s_acc)
        q_acc[...] = jnp.zeros_like(q_acc)

    s_acc[...] += jnp.sum(y, axis=0, keepdims=True)
    q_acc[...] += jnp.sum(y * y, axis=0, keepdims=True)

    @pl.when(i == _NB - 1)
    def _():
        st_ref[0] = s_acc[0]
        st_ref[1] = q_acc[0]


def _d1_call(deg2, x, acc1, acc2, W, bias):
    return pl.pallas_call(
        _d1_body,
        grid=(_NB,),
        in_specs=[pl.BlockSpec((_BN, 2), lambda i: (i, 0)),
                  pl.BlockSpec((_BN, C), lambda i: (i, 0)),
                  pl.BlockSpec((NC, _BN, HALF), lambda i: (0, i, 0)),
                  pl.BlockSpec((NC, _BN, HALF), lambda i: (0, i, 0)),
                  pl.BlockSpec((3, C, C), lambda i: (0, 0, 0)),
                  pl.BlockSpec((C,), lambda i: (0,))],
        out_specs=[pl.BlockSpec((_BN, C), lambda i: (i, 0)),
                   pl.BlockSpec((2, C), lambda i: (0, 0))],
        out_shape=[jax.ShapeDtypeStruct((N, C), jnp.float32),
                   jax.ShapeDtypeStruct((2, C), jnp.float32)],
        scratch_shapes=[pltpu.VMEM((1, C), jnp.float32),
                        pltpu.VMEM((1, C), jnp.float32)],
    )(deg2, x, acc1, acc2, W, bias)


def _d2_body(st_ref, x_ref, y_ref, g_ref, bt_ref, o_ref):
    mean = st_ref[0] / N
    var = st_ref[1] / N - mean * mean
    scale = (lax.rsqrt(var + EPS) * g_ref[...])[None, :]
    o_ref[...] = jnp.maximum(
        (y_ref[...] - mean[None, :]) * scale + bt_ref[...][None, :] + x_ref[...],
        0.0)


def _d2_call(st, x, y, gamma, beta):
    return pl.pallas_call(
        _d2_body,
        grid=(_NB,),
        in_specs=[pl.BlockSpec((2, C), lambda i: (0, 0)),
                  pl.BlockSpec((_BN, C), lambda i: (i, 0)),
                  pl.BlockSpec((_BN, C), lambda i: (i, 0)),
                  pl.BlockSpec((C,), lambda i: (0,)),
                  pl.BlockSpec((C,), lambda i: (0,))],
        out_specs=pl.BlockSpec((_BN, C), lambda i: (i, 0)),
        out_shape=jax.ShapeDtypeStruct((N, C), jnp.float32),
    )(st, x, y, gamma, beta)


def kernel(x, edge_index, W, bias, gamma, beta):
    row = edge_index[0]
    col = edge_index[1]
    degs = _deg_call(row)
    deg2 = degs[:, :N, 0].T
    hp0 = _e1_call(deg2, x).reshape(NC * NPAD, HALF)
    acc1 = _prop_call(hp0, row, col)
    hp1 = _e2_call(deg2, acc1).reshape(NC * NPAD, HALF)
    acc2 = _prop_call(hp1, row, col)
    y, st = _d1_call(deg2, x, acc1, acc2, W, bias)
    return _d2_call(st, x, y, gamma, beta)
